# baseline (speedup 1.0000x reference)
import functools

import jax
import jax.numpy as jnp
from jax import lax
from jax.experimental import pallas as pl
from jax.experimental.pallas import tpu as pltpu
from jax.experimental.pallas import tpu_sc as plsc

C = 4
N_NODES = 10000
N_EDGES = 320000
FEAT = 128

NC = 2
NS = 16
NW = NC * NS


def _k1_body(x_ref, wt_ref, o_ref):
    o_ref[...] = jnp.dot(x_ref[...], wt_ref[...],
                         preferred_element_type=jnp.float32)


def _linear(x, W_lin):
    return pl.pallas_call(
        _k1_body,
        out_shape=jax.ShapeDtypeStruct((x.shape[0], W_lin.shape[0]), jnp.float32),
    )(x, W_lin.T)


EPW = N_EDGES // NW
K2_CH = 200
K2_NCH = EPW // K2_CH


def _k2_body(h, rowi, coli, a_out, b_out,
             rv, cv, av0, av1, bv0, bv1, gs0, gs1, ws0, ws1):
    wid = lax.axis_index("s") * NC + lax.axis_index("c")
    base = wid * EPW
    pltpu.sync_copy(rowi.at[pl.ds(base, EPW)], rv)
    pltpu.sync_copy(coli.at[pl.ds(base, EPW)], cv)
    av = [av0, av1]
    bv = [bv0, bv1]
    gs = [gs0, gs1]
    ws = [ws0, ws1]
    gdesc = [None, None]
    wdesc = [None, None]

    def fire(g, s):
        ia = rv.at[pl.ds(g * K2_CH, K2_CH)]
        ib = cv.at[pl.ds(g * K2_CH, K2_CH)]
        gdesc[s] = (pltpu.async_copy(h.at[ia], av[s], gs[s]),
                    pltpu.async_copy(h.at[ib], bv[s], gs[s]))

    def fire_writes(g, s):
        off = base + g * K2_CH
        wdesc[s] = (pltpu.async_copy(av[s], a_out.at[pl.ds(off, K2_CH)], ws[s]),
                    pltpu.async_copy(bv[s], b_out.at[pl.ds(off, K2_CH)], ws[s]))

    fire(0, 0)
    for g in range(K2_NCH):
        s = g & 1
        if g + 1 < K2_NCH:
            if wdesc[1 - s] is not None:
                for d in wdesc[1 - s]:
                    d.wait()
                wdesc[1 - s] = None
            fire(g + 1, 1 - s)
        for d in gdesc[s]:
            d.wait()
        fire_writes(g, s)
    for s in (0, 1):
        if wdesc[s] is not None:
            for d in wdesc[s]:
                d.wait()


def _edge_gather(h, row, col):
    mesh = plsc.VectorSubcoreMesh(core_axis_name="c", subcore_axis_name="s",
                                  num_cores=NC, num_subcores=NS)
    f = pl.kernel(
        _k2_body,
        out_type=(jax.ShapeDtypeStruct((N_EDGES, FEAT), jnp.float32),
                  jax.ShapeDtypeStruct((N_EDGES, FEAT), jnp.float32)),
        mesh=mesh,
        scratch_types=[
            pltpu.VMEM((EPW,), jnp.int32),
            pltpu.VMEM((EPW,), jnp.int32),
            pltpu.VMEM((K2_CH, FEAT), jnp.float32),
            pltpu.VMEM((K2_CH, FEAT), jnp.float32),
            pltpu.VMEM((K2_CH, FEAT), jnp.float32),
            pltpu.VMEM((K2_CH, FEAT), jnp.float32),
            pltpu.SemaphoreType.DMA,
            pltpu.SemaphoreType.DMA,
            pltpu.SemaphoreType.DMA,
            pltpu.SemaphoreType.DMA,
        ],
    )
    return f(h, row, col)


K3_BE = 2560
K3_NB = N_EDGES // K3_BE


def _k3_body(a_ref, b_ref, watt_ref, w_ref):
    r = jnp.maximum(0.5 * a_ref[...] + b_ref[...], 0.0)
    logits = lax.dot_general(watt_ref[...], r,
                             (((1,), (1,)), ((), ())),
                             preferred_element_type=jnp.float32)
    m = jnp.max(logits, axis=0, keepdims=True)
    e = jnp.exp(logits - m)
    w_ref[...] = e / jnp.sum(e, axis=0, keepdims=True)


def _edge_attention(a, b, W_att):
    return pl.pallas_call(
        _k3_body,
        grid=(K3_NB,),
        in_specs=[
            pl.BlockSpec((K3_BE, FEAT), lambda i: (i, 0)),
            pl.BlockSpec((K3_BE, FEAT), lambda i: (i, 0)),
            pl.BlockSpec((C, FEAT), lambda i: (0, 0)),
        ],
        out_specs=pl.BlockSpec((C, K3_BE), lambda i: (0, i)),
        out_shape=jax.ShapeDtypeStruct((C, N_EDGES), jnp.float32),
    )(a, b, W_att)


NV = 64
NPV = 160
OUT_PAD = NV * NPV
K4_SCH = 3200
K4_NCH = N_EDGES // K4_SCH
K4_GB = 80


def _k4_body(rowi, coli, w0, w1, w2, w3, h, out,
             acc0, acc1, acc2, acc3,
             rows0, rows1, cols0, cols1,
             pe0, pe1, pc0, pc1,
             bbuf0, bbuf1, wbuf0, wbuf1,
             gsem0, gsem1, isem0, isem1):
    ws = [w0, w1, w2, w3]
    accs = [acc0, acc1, acc2, acc3]
    wid = lax.axis_index("s") * NC + lax.axis_index("c")
    rows = [rows0, rows1]
    cols = [cols0, cols1]
    pe = [pe0, pe1]
    pc = [pc0, pc1]
    bbuf = [bbuf0, bbuf1]
    wbuf = [wbuf0, wbuf1]
    gsem = [gsem0, gsem1]
    isem = [isem0, isem1]
    zi16 = jnp.zeros((16,), jnp.int32)
    zf16 = jnp.zeros((16,), jnp.float32)
    iota16 = lax.iota(jnp.int32, 16)

    def zpend(i, carry):
        for s in range(2):
            pe[s][pl.ds(i * 16, 16)] = zi16
            pc[s][pl.ds(i * 16, 16)] = zi16
        return carry
    lax.fori_loop(0, (K4_SCH + 16) // 16, zpend, 0)

    def fire_idx(g, s):
        pltpu.async_copy(rowi.at[pl.ds(g * K4_SCH, K4_SCH)], rows[s], isem[s])
        pltpu.async_copy(coli.at[pl.ds(g * K4_SCH, K4_SCH)], cols[s], isem[s])

    def wait_idx(g, s):
        pltpu.make_async_copy(rowi.at[pl.ds(g * K4_SCH, K4_SCH)], rows[s],
                              isem[s]).wait()
        pltpu.make_async_copy(coli.at[pl.ds(g * K4_SCH, K4_SCH)], cols[s],
                              isem[s]).wait()

    def fire_grp(sp, sb, q0, gc):
        nb = (gc + 15) // 16

        def fire_g(k, carry2):
            idc = pc[sp][pl.ds(q0 + k * 16, 16)] & 0x3FFF
            ide = pe[sp][pl.ds(q0 + k * 16, 16)]
            pltpu.async_copy(h.at[idc], bbuf[sb].at[pl.ds(k * 16, 16)],
                             gsem[sb])
            for c in range(C):
                pltpu.async_copy(ws[c].at[ide],
                                 wbuf[sb].at[c, pl.ds(k * 16, 16)], gsem[sb])
            return carry2
        lax.fori_loop(0, nb, fire_g, 0)

    def drain_grp(sp, sb, q0, gc):
        nb = (gc + 15) // 16

        def drain_g(k, carry2):
            idc = pc[sp][pl.ds(q0 + k * 16, 16)] & 0x3FFF
            ide = pe[sp][pl.ds(q0 + k * 16, 16)]
            pltpu.make_async_copy(h.at[idc], bbuf[sb].at[pl.ds(k * 16, 16)],
                                  gsem[sb]).wait()
            for c in range(C):
                pltpu.make_async_copy(ws[c].at[ide],
                                      wbuf[sb].at[c, pl.ds(k * 16, 16)],
                                      gsem[sb]).wait()
            return carry2
        lax.fori_loop(0, nb, drain_g, 0)

    def accum_grp(sp, sb, q0, gc):
        def load_edge(j):
            lr = pc[sp][pl.ds(q0 + j, 16)][0] >> 14
            w_ = [wbuf[sb][c, pl.ds(j, 16)][0] for c in range(C)]
            return lr, w_[0], w_[1], w_[2], w_[3]

        def accum(j, carry2):
            lr, wc0, wc1, wc2, wc3 = carry2
            nxt = load_edge(j + 1)
            wcs = [wc0, wc1, wc2, wc3]
            bvecs = [bbuf[sb][j, pl.ds(k * 16, 16)] for k in range(8)]
            for c in range(C):
                for k in range(8):
                    plsc.addupdate(accs[c].at[lr, pl.ds(k * 16, 16)],
                                   wcs[c] * bvecs[k])
            return nxt
        lax.fori_loop(0, gc, accum, load_edge(0))

    for p in range(2):
        vw = wid + p * NW
        n0 = vw * NPV

        def zacc(r, carry):
            for c in range(C):
                for k in range(8):
                    accs[c][r, pl.ds(k * 16, 16)] = zf16
            return carry
        lax.fori_loop(0, NPV, zacc, 0)

        fire_idx(0, 0)

        def chunk(g, s, carry):
            q0l, gcl = carry
            off = g * K4_SCH
            wait_idx(g, s)

            @pl.when(g + 1 < K4_NCH)
            def _():
                fire_idx(g + 1, 1 - s)

            def scanv(v, cnt):
                vr = rows[s][pl.ds(v * 16, 16)]
                vc = cols[s][pl.ds(v * 16, 16)]
                msk = (vr >= n0) & (vr < n0 + NPV)
                veid = off + v * 16 + iota16
                m = plsc.all_reduce_population_count(msk).reshape(-1)[0]

                @pl.when(m > 0)
                def _():
                    incl = plsc.cumsum(jnp.where(msk, 1, 0))
                    pos = cnt + incl - 1
                    packed = vc | ((vr - n0) << 14)
                    plsc.store_scatter(pe[s], [pos], veid, mask=msk)
                    plsc.store_scatter(pc[s], [pos], packed, mask=msk)
                return cnt + m
            cnt = lax.fori_loop(0, K4_SCH // 16, scanv, jnp.int32(0),
                                unroll=2)

            @pl.when(gcl > 0)
            def _():
                drain_grp(1 - s, 1 - s, q0l, gcl)
                accum_grp(1 - s, 1 - s, q0l, gcl)

            ng = (cnt + K4_GB - 1) // K4_GB

            def early(q, carry2):
                eq0 = q * K4_GB
                fire_grp(s, 1 - s, eq0, K4_GB)
                drain_grp(s, 1 - s, eq0, K4_GB)
                accum_grp(s, 1 - s, eq0, K4_GB)
                return carry2
            lax.fori_loop(0, jnp.maximum(ng - 1, 0), early, 0)

            q0n = jnp.maximum(ng - 1, 0) * K4_GB
            gcn = cnt - q0n

            @pl.when(gcn > 0)
            def _():
                fire_grp(s, s, q0n, gcn)
            return q0n, gcn

        def super_chunk(gs, carry):
            carry = chunk(gs * 2, 0, carry)
            carry = chunk(gs * 2 + 1, 1, carry)
            return carry
        q0f, gcf = lax.fori_loop(0, K4_NCH // 2, super_chunk,
                                 (jnp.int32(0), jnp.int32(0)))

        @pl.when(gcf > 0)
        def _():
            drain_grp(1, 1, q0f, gcf)
            accum_grp(1, 1, q0f, gcf)

        for c in range(C):
            pltpu.sync_copy(
                accs[c],
                out.at[pl.ds(n0, NPV), pl.ds(c * FEAT, FEAT)])


def _aggregate(row, col, w0, w1, w2, w3, h):
    mesh = plsc.VectorSubcoreMesh(core_axis_name="c", subcore_axis_name="s",
                                  num_cores=NC, num_subcores=NS)
    f = pl.kernel(
        _k4_body,
        out_type=jax.ShapeDtypeStruct((OUT_PAD, C * FEAT), jnp.float32),
        mesh=mesh,
        compiler_params=pltpu.CompilerParams(needs_layout_passes=False),
        scratch_types=(
            [pltpu.VMEM((NPV, FEAT), jnp.float32)] * 4
            + [pltpu.VMEM((K4_SCH,), jnp.int32)] * 4
            + [pltpu.VMEM((K4_SCH + 16,), jnp.int32)] * 4
            + [pltpu.VMEM((K4_GB, FEAT), jnp.float32)] * 2
            + [pltpu.VMEM((C, K4_GB + 16), jnp.float32)] * 2
            + [pltpu.SemaphoreType.DMA] * 4
        ),
    )
    return f(row, col, w0, w1, w2, w3, h)


def kernel(x, edge_index, W_lin, W_att):
    h = _linear(x, W_lin)
    row = edge_index[0]
    col = edge_index[1]
    a, b = _edge_gather(h, row, col)
    w = _edge_attention(a, b, W_att)
    outp = _aggregate(row, col, w[0], w[1], w[2], w[3], h)
    return outp[:N_NODES]


# speedup vs baseline: 1.3250x; 1.3250x over previous
import functools

import jax
import jax.numpy as jnp
from jax import lax
from jax.experimental import pallas as pl
from jax.experimental.pallas import tpu as pltpu
from jax.experimental.pallas import tpu_sc as plsc

C = 4
N_NODES = 10000
N_EDGES = 320000
FEAT = 128

NC = 2
NS = 16
NW = NC * NS


def _k1_body(x_ref, wt_ref, o_ref):
    o_ref[...] = jnp.dot(x_ref[...], wt_ref[...],
                         preferred_element_type=jnp.float32)


def _linear(x, W_lin):
    return pl.pallas_call(
        _k1_body,
        out_shape=jax.ShapeDtypeStruct((x.shape[0], W_lin.shape[0]), jnp.float32),
    )(x, W_lin.T)


EPW = N_EDGES // NW
K2_CH = 200
K2_NCH = EPW // K2_CH


def _k2_body(h, rowi, coli, a_out, b_out,
             rv, cv, av0, av1, bv0, bv1, gs0, gs1, ws0, ws1):
    wid = lax.axis_index("s") * NC + lax.axis_index("c")
    base = wid * EPW
    pltpu.sync_copy(rowi.at[pl.ds(base, EPW)], rv)
    pltpu.sync_copy(coli.at[pl.ds(base, EPW)], cv)
    av = [av0, av1]
    bv = [bv0, bv1]
    gs = [gs0, gs1]
    ws = [ws0, ws1]
    gdesc = [None, None]
    wdesc = [None, None]

    def fire(g, s):
        ia = rv.at[pl.ds(g * K2_CH, K2_CH)]
        ib = cv.at[pl.ds(g * K2_CH, K2_CH)]
        gdesc[s] = (pltpu.async_copy(h.at[ia], av[s], gs[s]),
                    pltpu.async_copy(h.at[ib], bv[s], gs[s]))

    def fire_writes(g, s):
        off = base + g * K2_CH
        wdesc[s] = (pltpu.async_copy(av[s], a_out.at[pl.ds(off, K2_CH)], ws[s]),
                    pltpu.async_copy(bv[s], b_out.at[pl.ds(off, K2_CH)], ws[s]))

    fire(0, 0)
    for g in range(K2_NCH):
        s = g & 1
        if g + 1 < K2_NCH:
            if wdesc[1 - s] is not None:
                for d in wdesc[1 - s]:
                    d.wait()
                wdesc[1 - s] = None
            fire(g + 1, 1 - s)
        for d in gdesc[s]:
            d.wait()
        fire_writes(g, s)
    for s in (0, 1):
        if wdesc[s] is not None:
            for d in wdesc[s]:
                d.wait()


def _edge_gather(h, row, col):
    mesh = plsc.VectorSubcoreMesh(core_axis_name="c", subcore_axis_name="s",
                                  num_cores=NC, num_subcores=NS)
    f = pl.kernel(
        _k2_body,
        out_type=(jax.ShapeDtypeStruct((N_EDGES, FEAT), jnp.float32),
                  jax.ShapeDtypeStruct((N_EDGES, FEAT), jnp.float32)),
        mesh=mesh,
        scratch_types=[
            pltpu.VMEM((EPW,), jnp.int32),
            pltpu.VMEM((EPW,), jnp.int32),
            pltpu.VMEM((K2_CH, FEAT), jnp.float32),
            pltpu.VMEM((K2_CH, FEAT), jnp.float32),
            pltpu.VMEM((K2_CH, FEAT), jnp.float32),
            pltpu.VMEM((K2_CH, FEAT), jnp.float32),
            pltpu.SemaphoreType.DMA,
            pltpu.SemaphoreType.DMA,
            pltpu.SemaphoreType.DMA,
            pltpu.SemaphoreType.DMA,
        ],
    )
    return f(h, row, col)


K3_BE = 2560
K3_NB = N_EDGES // K3_BE


def _k3_body(a_ref, b_ref, watt_ref, w_ref):
    r = jnp.maximum(0.5 * a_ref[...] + b_ref[...], 0.0)
    logits = lax.dot_general(watt_ref[...], r,
                             (((1,), (1,)), ((), ())),
                             preferred_element_type=jnp.float32)
    m = jnp.max(logits, axis=0, keepdims=True)
    e = jnp.exp(logits - m)
    w_ref[...] = e / jnp.sum(e, axis=0, keepdims=True)


def _edge_attention(a, b, W_att):
    return pl.pallas_call(
        _k3_body,
        grid=(K3_NB,),
        in_specs=[
            pl.BlockSpec((K3_BE, FEAT), lambda i: (i, 0)),
            pl.BlockSpec((K3_BE, FEAT), lambda i: (i, 0)),
            pl.BlockSpec((C, FEAT), lambda i: (0, 0)),
        ],
        out_specs=pl.BlockSpec((C, K3_BE), lambda i: (0, i)),
        out_shape=jax.ShapeDtypeStruct((C, N_EDGES), jnp.float32),
    )(a, b, W_att)


NV = 64
NPV = 160
OUT_PAD = NV * NPV
K4_SCH = 3200
K4_NCH = N_EDGES // K4_SCH
K4_GB = 80


def _k4_body(rowi, coli, w0, w1, w2, w3, h, out,
             acc0, acc1, acc2, acc3,
             rows0, rows1, cols0, cols1,
             pe0, pe1, pc0, pc1,
             bbuf0, bbuf1, wbuf0, wbuf1,
             gsem0, gsem1, isem0, isem1):
    ws = [w0, w1, w2, w3]
    accs = [acc0, acc1, acc2, acc3]
    wid = lax.axis_index("s") * NC + lax.axis_index("c")
    rows = [rows0, rows1]
    cols = [cols0, cols1]
    pe = [pe0, pe1]
    pc = [pc0, pc1]
    bbuf = [bbuf0, bbuf1]
    wbuf = [wbuf0, wbuf1]
    gsem = [gsem0, gsem1]
    isem = [isem0, isem1]
    zi16 = jnp.zeros((16,), jnp.int32)
    zf16 = jnp.zeros((16,), jnp.float32)
    iota16 = lax.iota(jnp.int32, 16)

    def zpend(i, carry):
        for s in range(2):
            pe[s][pl.ds(i * 16, 16)] = zi16
            pc[s][pl.ds(i * 16, 16)] = zi16
        return carry
    lax.fori_loop(0, (K4_SCH + 16) // 16, zpend, 0)

    def fire_idx(g, s):
        pltpu.async_copy(rowi.at[pl.ds(g * K4_SCH, K4_SCH)], rows[s], isem[s])
        pltpu.async_copy(coli.at[pl.ds(g * K4_SCH, K4_SCH)], cols[s], isem[s])

    def wait_idx(g, s):
        pltpu.make_async_copy(rowi.at[pl.ds(g * K4_SCH, K4_SCH)], rows[s],
                              isem[s]).wait()
        pltpu.make_async_copy(coli.at[pl.ds(g * K4_SCH, K4_SCH)], cols[s],
                              isem[s]).wait()

    def fire_grp(sp, sb, q0, gc):
        nb = (gc + 15) // 16

        def fire_g(k, carry2):
            idc = pc[sp][pl.ds(q0 + k * 16, 16)] & 0x3FFF
            ide = pe[sp][pl.ds(q0 + k * 16, 16)]
            pltpu.async_copy(h.at[idc], bbuf[sb].at[pl.ds(k * 16, 16)],
                             gsem[sb])
            for c in range(C):
                pltpu.async_copy(ws[c].at[ide],
                                 wbuf[sb].at[c, pl.ds(k * 16, 16)], gsem[sb])
            return carry2
        lax.fori_loop(0, nb, fire_g, 0)

    def drain_grp(sp, sb, q0, gc):
        nb = (gc + 15) // 16

        def drain_g(k, carry2):
            idc = pc[sp][pl.ds(q0 + k * 16, 16)] & 0x3FFF
            ide = pe[sp][pl.ds(q0 + k * 16, 16)]
            pltpu.make_async_copy(h.at[idc], bbuf[sb].at[pl.ds(k * 16, 16)],
                                  gsem[sb]).wait()
            for c in range(C):
                pltpu.make_async_copy(ws[c].at[ide],
                                      wbuf[sb].at[c, pl.ds(k * 16, 16)],
                                      gsem[sb]).wait()
            return carry2
        lax.fori_loop(0, nb, drain_g, 0)

    def accum_grp(sp, sb, q0, gc):
        def load_edge(j):
            lr = pc[sp][pl.ds(q0 + j, 16)][0] >> 14
            w_ = [wbuf[sb][c, pl.ds(j, 16)][0] for c in range(C)]
            return lr, w_[0], w_[1], w_[2], w_[3]

        def accum(j, carry2):
            lr, wc0, wc1, wc2, wc3 = carry2
            nxt = load_edge(j + 1)
            wcs = [wc0, wc1, wc2, wc3]
            bvecs = [bbuf[sb][j, pl.ds(k * 16, 16)] for k in range(8)]
            for c in range(C):
                for k in range(8):
                    plsc.addupdate(accs[c].at[lr, pl.ds(k * 16, 16)],
                                   wcs[c] * bvecs[k])
            return nxt
        lax.fori_loop(0, gc, accum, load_edge(0))

    for p in range(2):
        vw = wid + p * NW
        n0 = vw * NPV

        def zacc(r, carry):
            for c in range(C):
                for k in range(8):
                    accs[c][r, pl.ds(k * 16, 16)] = zf16
            return carry
        lax.fori_loop(0, NPV, zacc, 0)

        fire_idx(0, 0)

        def chunk(g, s, carry):
            q0l, gcl = carry
            off = g * K4_SCH
            wait_idx(g, s)

            @pl.when(g + 1 < K4_NCH)
            def _():
                fire_idx(g + 1, 1 - s)

            def scanv(v, cnt):
                vr = rows[s][pl.ds(v * 16, 16)]
                vc = cols[s][pl.ds(v * 16, 16)]
                msk = (vr >= n0) & (vr < n0 + NPV)
                veid = off + v * 16 + iota16
                incl = plsc.cumsum(jnp.where(msk, 1, 0))
                pos = cnt + incl - 1
                packed = vc | ((vr - n0) << 14)
                plsc.store_scatter(pe[s], [pos], veid, mask=msk)
                plsc.store_scatter(pc[s], [pos], packed, mask=msk)
                m = plsc.all_reduce_population_count(msk)
                return cnt + m.reshape(-1)[0]
            cnt = lax.fori_loop(0, K4_SCH // 16, scanv, jnp.int32(0),
                                unroll=2)

            @pl.when(gcl > 0)
            def _():
                drain_grp(1 - s, 1 - s, q0l, gcl)
                accum_grp(1 - s, 1 - s, q0l, gcl)

            ng = (cnt + K4_GB - 1) // K4_GB

            def early(q, carry2):
                eq0 = q * K4_GB
                fire_grp(s, 1 - s, eq0, K4_GB)
                drain_grp(s, 1 - s, eq0, K4_GB)
                accum_grp(s, 1 - s, eq0, K4_GB)
                return carry2
            lax.fori_loop(0, jnp.maximum(ng - 1, 0), early, 0)

            q0n = jnp.maximum(ng - 1, 0) * K4_GB
            gcn = cnt - q0n

            @pl.when(gcn > 0)
            def _():
                fire_grp(s, s, q0n, gcn)
            return q0n, gcn

        def super_chunk(gs, carry):
            carry = chunk(gs * 2, 0, carry)
            carry = chunk(gs * 2 + 1, 1, carry)
            return carry
        q0f, gcf = lax.fori_loop(0, K4_NCH // 2, super_chunk,
                                 (jnp.int32(0), jnp.int32(0)))

        @pl.when(gcf > 0)
        def _():
            drain_grp(1, 1, q0f, gcf)
            accum_grp(1, 1, q0f, gcf)

        for c in range(C):
            pltpu.sync_copy(
                accs[c],
                out.at[pl.ds(n0, NPV), pl.ds(c * FEAT, FEAT)])


def _aggregate(row, col, w0, w1, w2, w3, h):
    mesh = plsc.VectorSubcoreMesh(core_axis_name="c", subcore_axis_name="s",
                                  num_cores=NC, num_subcores=NS)
    f = pl.kernel(
        _k4_body,
        out_type=jax.ShapeDtypeStruct((OUT_PAD, C * FEAT), jnp.float32),
        mesh=mesh,
        compiler_params=pltpu.CompilerParams(needs_layout_passes=False),
        scratch_types=(
            [pltpu.VMEM((NPV, FEAT), jnp.float32)] * 4
            + [pltpu.VMEM((K4_SCH,), jnp.int32)] * 4
            + [pltpu.VMEM((K4_SCH + 16,), jnp.int32)] * 4
            + [pltpu.VMEM((K4_GB, FEAT), jnp.float32)] * 2
            + [pltpu.VMEM((C, K4_GB + 16), jnp.float32)] * 2
            + [pltpu.SemaphoreType.DMA] * 4
        ),
    )
    return f(row, col, w0, w1, w2, w3, h)


def kernel(x, edge_index, W_lin, W_att):
    h = _linear(x, W_lin)
    row = edge_index[0]
    col = edge_index[1]
    a, b = _edge_gather(h, row, col)
    w = _edge_attention(a, b, W_att)
    outp = _aggregate(row, col, w[0], w[1], w[2], w[3], h)
    return outp[:N_NODES]
